# Initial kernel scaffold; baseline (speedup 1.0000x reference)
import jax
import jax.numpy as jnp
from jax.experimental import pallas as pl


def kernel(x, R_inv, mb_table, mb_W1, mb_b1, mb_W2, mb_b2, grey_table, g_W1, g_b1, g_W2, g_b2):
    raise NotImplementedError("write your pallas kernel here")


# speedup vs baseline: 152.7198x; 152.7198x over previous
import functools

import numpy as np
import jax
import jax.numpy as jnp
from jax import lax
from jax.experimental import pallas as pl
from jax.experimental.pallas import tpu as pltpu
from jax.experimental.pallas import tpu_sc as plsc

SCALE = 0.5
L = 8
F = 2
T = 1 << 19
N_MIN = 32
_B = float(np.exp(np.log(2048.0 * SCALE / N_MIN) / (L - 1)))
RES = tuple(int(np.floor(N_MIN * _B ** l)) for l in range(L))
DENSE = tuple(r ** 3 <= T for r in RES)
P1 = np.int32(np.uint32(2654435761).view(np.int32))
P2 = np.int32(805459861)
MASK = np.int32(T - 1)

C = 64
RW = 16
NW = 32
G16 = C // 16


@functools.lru_cache(maxsize=None)
def _sc_encode(N):
    PW = N // NW
    NCH = PW // C
    mesh = plsc.VectorSubcoreMesh(core_axis_name="c", subcore_axis_name="s")
    scratch = (
        [pltpu.VMEM((C,), jnp.float32) for _ in range(3)]
        + [pltpu.VMEM((32, C), jnp.float32)]
        + [pltpu.VMEM((8 * C,), jnp.int32) for _ in range(L)]
        + [pltpu.VMEM((8 * C, RW), jnp.float32) for _ in range(L)]
        + [pltpu.SemaphoreType.DMA]
    )

    @functools.partial(
        pl.kernel,
        mesh=mesh,
        out_type=jax.ShapeDtypeStruct((32, N), jnp.float32),
        scratch_types=scratch,
        compiler_params=pltpu.CompilerParams(
            needs_layout_passes=False, use_tc_tiling_on_sc=False),
    )
    def enc(x0_hbm, x1_hbm, x2_hbm, t0, t1, t2, t3, t4, t5, t6, t7,
            feats_hbm, xv0, xv1, xv2, feats_v,
            i0, i1, i2, i3, i4, i5, i6, i7,
            r0, r1, r2, r3, r4, r5, r6, r7, sem):
        tabs = (t0, t1, t2, t3, t4, t5, t6, t7)
        idxs = (i0, i1, i2, i3, i4, i5, i6, i7)
        rows = (r0, r1, r2, r3, r4, r5, r6, r7)
        xvs = (xv0, xv1, xv2)
        xhs = (x0_hbm, x1_hbm, x2_hbm)
        wid = lax.axis_index("s") * 2 + lax.axis_index("c")
        IOTA = lax.iota(jnp.int32, 16)
        FC = tuple(jnp.full((16,), f, jnp.int32) for f in range(4))

        def pass_a(g, carry):
            off = g * 16
            xs = tuple(v[pl.ds(off, 16)] for v in xvs)
            for l in range(L):
                r = RES[l]
                s = np.float32(r - 1)
                p0 = tuple((x * s).astype(jnp.int32) for x in xs)
                c1 = tuple(jnp.minimum(q + 1, r - 1) for q in p0)
                if DENSE[l]:
                    A = (p0[0] * (r * r), c1[0] * (r * r))
                    Bv = (p0[1] * r, c1[1] * r)
                    K = (p0[2], c1[2])
                else:
                    A = (p0[0], c1[0])
                    Bv = (p0[1] * P1, c1[1] * P1)
                    K = (p0[2] * P2, c1[2] * P2)
                ci = 0
                for i in range(2):
                    for j in range(2):
                        for k in range(2):
                            if DENSE[l]:
                                idx = A[i] + Bv[j] + K[k]
                            else:
                                idx = (A[i] ^ Bv[j] ^ K[k]) & MASK
                            idxs[l][pl.ds(ci * C + off, 16)] = idx
                            ci += 1
            return carry

        def pass_b(g, carry):
            off = g * 16
            xs = tuple(v[pl.ds(off, 16)] for v in xvs)
            roff = IOTA + off
            for l in range(L):
                r = RES[l]
                s = np.float32(r - 1)
                pos = tuple(x * s for x in xs)
                p0 = tuple(q.astype(jnp.int32) for q in pos)
                w = tuple(q - f.astype(jnp.float32) for q, f in zip(pos, p0))
                u = tuple((1.0 - wd, wd) for wd in w)
                wij = [[u[0][i] * u[1][j] for j in range(2)] for i in range(2)]
                acc = None
                ci = 0
                for i in range(2):
                    for j in range(2):
                        for k in range(2):
                            wc = wij[i][j] * u[2][k]
                            ridx = roff + (ci * C)
                            fs = tuple(
                                plsc.load_gather(rows[l], [ridx, FC[f]])
                                for f in range(4)
                            )
                            if acc is None:
                                acc = [wc * f for f in fs]
                            else:
                                acc = [a + wc * f for a, f in zip(acc, fs)]
                            ci += 1
                feats_v[2 * l, pl.ds(off, 16)] = acc[0]
                feats_v[2 * l + 1, pl.ds(off, 16)] = acc[1]
                feats_v[16 + 2 * l, pl.ds(off, 16)] = acc[2]
                feats_v[17 + 2 * l, pl.ds(off, 16)] = acc[3]
            return carry

        def chunk(tch, carry):
            nbase = wid * PW + tch * C
            for xh, xv in zip(xhs, xvs):
                pltpu.sync_copy(xh.at[pl.ds(nbase, C)], xv)
            lax.fori_loop(0, G16, pass_a, 0)
            handles = [
                pltpu.async_copy(tabs[l].at[idxs[l]], rows[l], sem)
                for l in range(L)
            ]
            for h in handles:
                h.wait()
            lax.fori_loop(0, G16, pass_b, 0)
            pltpu.sync_copy(feats_v, feats_hbm.at[:, pl.ds(nbase, C)])
            return carry

        lax.fori_loop(0, NCH, chunk, 0)

    return enc


@functools.lru_cache(maxsize=None)
def _mlp(N):
    BLK = 8192

    def body(h_ref, w1_ref, b1_ref, w2_ref, b2_ref, o_ref):
        h = h_ref[...]
        hid = jnp.dot(w1_ref[...], h, preferred_element_type=jnp.float32,
                      precision=lax.Precision.HIGHEST) + b1_ref[...]
        hid = jnp.maximum(hid, 0.0)
        o = jnp.dot(w2_ref[...], hid, preferred_element_type=jnp.float32,
                    precision=lax.Precision.HIGHEST) + b2_ref[...]
        o_ref[...] = 1.0 / (1.0 + jnp.exp(-o))

    return pl.pallas_call(
        body,
        grid=(N // BLK,),
        in_specs=[
            pl.BlockSpec((32, BLK), lambda i: (0, i)),
            pl.BlockSpec((64, 32), lambda i: (0, 0)),
            pl.BlockSpec((64, 1), lambda i: (0, 0)),
            pl.BlockSpec((2, 64), lambda i: (0, 0)),
            pl.BlockSpec((2, 1), lambda i: (0, 0)),
        ],
        out_specs=pl.BlockSpec((2, BLK), lambda i: (0, i)),
        out_shape=jax.ShapeDtypeStruct((2, N), jnp.float32),
    )


def kernel(x, R_inv, mb_table, mb_W1, mb_b1, mb_W2, mb_b2,
           grey_table, g_W1, g_b1, g_W2, g_b2):
    N = x.shape[0]
    xr = x @ R_inv.T
    xn = (xr + SCALE) / (2.0 * SCALE)
    xcols = tuple(xn[:, d] for d in range(3))
    pad = jnp.zeros((T, RW - 2 * F), jnp.float32)
    tabs = [jnp.concatenate([mb_table[l], grey_table[l], pad], axis=-1)
            for l in range(L)]
    feats = _sc_encode(N)(*xcols, *tabs)

    z = jnp.zeros((32, 16), jnp.float32)
    W1T = jnp.concatenate(
        [jnp.concatenate([mb_W1.T, z], 1), jnp.concatenate([z, g_W1.T], 1)], 0)
    b1c = jnp.concatenate([mb_b1, g_b1]).reshape(64, 1)
    z2 = jnp.zeros((1, 32), jnp.float32)
    W2T = jnp.concatenate(
        [jnp.concatenate([mb_W2.T, z2], 1), jnp.concatenate([z2, g_W2.T], 1)], 0)
    b2c = jnp.stack([mb_b2[0], g_b2[0]]).reshape(2, 1)

    out = _mlp(N)(feats, W1T, b1c, W2T, b2c)
    return out[0], out[1]
